# Initial kernel scaffold; baseline (speedup 1.0000x reference)
import functools

import jax
import jax.numpy as jnp
from jax import lax
from jax.experimental import pallas as pl
from jax.experimental.pallas import tpu as pltpu
from jax.experimental.pallas import tpu_sc as plsc

_NC = 2
_NS = 16
_DW = 16


def _make_segsum(n, d, e, want_deg):
    nw = _NC * _NS
    ew = e // nw
    k = 80
    nch = ew // k
    assert ew * nw == e and nch * k == ew
    rpt = n // _NS
    zr = rpt // 5
    assert zr * 5 == rpt
    nwr = 10
    wr = n // nwr
    assert wr * nwr == n

    mesh = plsc.VectorSubcoreMesh(core_axis_name="c", subcore_axis_name="s")
    out_type = [jax.ShapeDtypeStruct((_NC * n, d), jnp.float32)]
    scratch = [
        pltpu.VMEM((k,), jnp.int32),
        pltpu.VMEM((k,), jnp.int32),
        pltpu.VMEM((k, d), jnp.float32),
        pltpu.VMEM((zr, d), jnp.float32),
        pltpu.VMEM_SHARED((n, d), jnp.float32),
        pltpu.SemaphoreType.DMA,
    ]
    if want_deg:
        out_type.append(jax.ShapeDtypeStruct((_NC * n, _DW), jnp.float32))
        scratch += [
            pltpu.VMEM((k, _DW), jnp.float32),
            pltpu.VMEM((zr, _DW), jnp.float32),
            pltpu.VMEM_SHARED((n, _DW), jnp.float32),
        ]

    def body(table_hbm, src_hbm, dst_hbm, *rest):
        if want_deg:
            (agg_out, deg_out, src_v, dst_v, rows_v, zrows, acc, sem,
             ones_v, zdeg, dacc) = rest
        else:
            agg_out, src_v, dst_v, rows_v, zrows, acc, sem = rest
        c = lax.axis_index("c")
        s = lax.axis_index("s")
        wid = c * _NS + s

        def zfill(r, carry):
            for jj in range(d // 16):
                zrows[r, pl.ds(jj * 16, 16)] = jnp.zeros((16,), jnp.float32)
            if want_deg:
                zdeg[r] = jnp.zeros((_DW,), jnp.float32)
            return carry
        lax.fori_loop(0, zr, zfill, 0)
        if want_deg:
            def ofill(r, carry):
                ones_v[r] = jnp.ones((_DW,), jnp.float32)
                return carry
            lax.fori_loop(0, k, ofill, 0)

        r0 = s * rpt
        for j in range(5):
            pltpu.sync_copy(zrows, acc.at[pl.ds(r0 + j * zr, zr)])
            if want_deg:
                pltpu.sync_copy(zdeg, dacc.at[pl.ds(r0 + j * zr, zr)])
        plsc.subcore_barrier()

        ebase = wid * ew

        def chunk(i, carry):
            base = ebase + i * k
            pltpu.sync_copy(src_hbm.at[pl.ds(base, k)], src_v)
            pltpu.sync_copy(dst_hbm.at[pl.ds(base, k)], dst_v)
            pltpu.async_copy(table_hbm.at[src_v], rows_v, sem).wait()
            pltpu.sync_copy(rows_v, acc.at[dst_v], add=True)
            if want_deg:
                pltpu.sync_copy(ones_v, dacc.at[dst_v], add=True)
            return carry
        lax.fori_loop(0, nch, chunk, 0)
        plsc.subcore_barrier()

        @pl.when(s < nwr)
        def _():
            w0 = s * wr
            pltpu.sync_copy(acc.at[pl.ds(w0, wr)],
                            agg_out.at[pl.ds(c * n + w0, wr)])
            if want_deg:
                pltpu.sync_copy(dacc.at[pl.ds(w0, wr)],
                                deg_out.at[pl.ds(c * n + w0, wr)])

    return pl.kernel(body, mesh=mesh, out_type=out_type, scratch_types=scratch)


def _dense_body(relu, agg_a, agg_b, deg_a, deg_b, x_ref, wlt, blr, wrt, out_ref):
    deg = jnp.maximum(deg_a[:, :1] + deg_b[:, :1], 1.0)
    mean = (agg_a[...] + agg_b[...]) / deg
    y = (jnp.dot(mean, wlt[...], preferred_element_type=jnp.float32)
         + blr[...]
         + jnp.dot(x_ref[...], wrt[...], preferred_element_type=jnp.float32))
    out_ref[...] = jnp.maximum(y, 0.0) if relu else y


def _dense(agg2, deg2, x, wlt, blr, wrt, relu):
    n, d = x.shape
    r = 2000
    nb = n // r
    assert nb * r == n
    grid = (nb,)
    return pl.pallas_call(
        functools.partial(_dense_body, relu),
        grid=grid,
        in_specs=[
            pl.BlockSpec((r, d), lambda i: (i, 0)),
            pl.BlockSpec((r, d), lambda i, _nb=nb: (i + _nb, 0)),
            pl.BlockSpec((r, _DW), lambda i: (i, 0)),
            pl.BlockSpec((r, _DW), lambda i, _nb=nb: (i + _nb, 0)),
            pl.BlockSpec((r, d), lambda i: (i, 0)),
            pl.BlockSpec((d, d), lambda i: (0, 0)),
            pl.BlockSpec((1, d), lambda i: (0, 0)),
            pl.BlockSpec((d, d), lambda i: (0, 0)),
        ],
        out_specs=pl.BlockSpec((r, d), lambda i: (i, 0)),
        out_shape=jax.ShapeDtypeStruct((n, d), jnp.float32),
    )(agg2, agg2, deg2, deg2, x, wlt, blr, wrt)


def kernel(x, edge_index, W1l, b1l, W1r, W2l, b2l, W2r):
    n, d = x.shape
    e = edge_index.shape[1]
    src = edge_index[0]
    dst = edge_index[1]

    segsum_deg = _make_segsum(n, d, e, want_deg=True)
    segsum = _make_segsum(n, d, e, want_deg=False)

    agg1, deg = segsum_deg(x, src, dst)
    h = _dense(agg1, deg, x, W1l.T, b1l.reshape(1, -1), W1r.T, relu=True)
    (agg2,) = segsum(h, src, dst)
    out = _dense(agg2, deg, h, W2l.T, b2l.reshape(1, -1), W2r.T, relu=False)
    return out


# speedup vs baseline: 5.3763x; 5.3763x over previous
import functools

import jax
import jax.numpy as jnp
from jax import lax
from jax.experimental import pallas as pl
from jax.experimental.pallas import tpu as pltpu
from jax.experimental.pallas import tpu_sc as plsc

_NC = 2
_NS = 16


def _make_segmean(n, d, e, layer1):
    nw = _NC * _NS
    ew = e // nw
    k = 80
    nch = ew // k
    assert ew * nw == e and nch * k == ew
    zr = 40
    nzc = n // zr
    assert zr * nzc == n
    nzi = (nzc + _NS - 1) // _NS
    ec = e // _NS
    cb_sz = 4000
    ncc = ec // cb_sz
    assert ncc * cb_sz == ec
    gch = 80
    ngc = n // gch
    ngi = (ngc + _NS - 1) // _NS

    mesh = plsc.VectorSubcoreMesh(core_axis_name="c", subcore_axis_name="s")
    out_type = [jax.ShapeDtypeStruct((_NC * n, d), jnp.float32)]
    scratch = [
        pltpu.VMEM((k,), jnp.int32),
        pltpu.VMEM((k,), jnp.int32),
        pltpu.VMEM((k, d), jnp.float32),
        pltpu.VMEM((zr, d), jnp.float32),
        pltpu.VMEM((zr,), jnp.float32),
        pltpu.VMEM_SHARED((n, d), jnp.float32),
        pltpu.SemaphoreType.DMA,
    ]
    if layer1:
        out_type.append(jax.ShapeDtypeStruct((n,), jnp.float32))
        scratch += [
            pltpu.VMEM((cb_sz,), jnp.int32),
            pltpu.VMEM((n,), jnp.float32),
            pltpu.VMEM((_NS * gch,), jnp.float32),
            pltpu.VMEM((gch,), jnp.float32),
            pltpu.VMEM_SHARED((_NS * n,), jnp.float32),
            pltpu.VMEM_SHARED((n,), jnp.float32),
        ]

    def body(*args):
        if layer1:
            (table_hbm, src_hbm, dst_hbm, agg_out, invdeg_out,
             src_v, dst_v, rows_v, zrows, dchunk, acc, sem,
             cntbuf, degloc, sumbuf, dsum, degstage, invdeg_sp) = args
        else:
            (table_hbm, src_hbm, dst_hbm, invdeg_hbm, agg_out,
             src_v, dst_v, rows_v, zrows, dchunk, acc, sem) = args
        c = lax.axis_index("c")
        s = lax.axis_index("s")
        wid = c * _NS + s

        def zfill(r, carry):
            for jj in range(d // 16):
                zrows[r, pl.ds(jj * 16, 16)] = jnp.zeros((16,), jnp.float32)
            return carry
        lax.fori_loop(0, zr, zfill, 0)

        def zinit(t, carry):
            idx = s + t * _NS
            @pl.when(idx < nzc)
            def _():
                pltpu.sync_copy(zrows, acc.at[pl.ds(idx * zr, zr)])
            return carry
        lax.fori_loop(0, nzi, zinit, 0)

        if layer1:
            def zdl(q, carry):
                degloc[pl.ds(q * 16, 16)] = jnp.zeros((16,), jnp.float32)
                return carry
            lax.fori_loop(0, n // 16, zdl, 0)
            ones16 = jnp.ones((16,), jnp.float32)
            cb = s * ec
            for cc in range(ncc):
                pltpu.sync_copy(dst_hbm.at[pl.ds(cb + cc * cb_sz, cb_sz)],
                                cntbuf)
                def cnt(q, carry):
                    iv = cntbuf[pl.ds(q * 16, 16)]
                    plsc.addupdate_scatter(degloc, [iv], ones16)
                    return carry
                lax.fori_loop(0, cb_sz // 16, cnt, 0)
            pltpu.sync_copy(degloc, degstage.at[pl.ds(s * n, n)])

        plsc.subcore_barrier()

        if layer1:
            def comb(t, carry):
                idx = s + t * _NS
                @pl.when(idx < ngc)
                def _():
                    g0 = idx * gch
                    for a in range(_NS):
                        pltpu.sync_copy(degstage.at[pl.ds(a * n + g0, gch)],
                                        sumbuf.at[pl.ds(a * gch, gch)])
                    for j in range(gch // 16):
                        tot = sumbuf[pl.ds(j * 16, 16)]
                        for a in range(1, _NS):
                            tot = tot + sumbuf[pl.ds(a * gch + j * 16, 16)]
                        inv = (jnp.ones((16,), jnp.float32)
                               / jnp.maximum(tot, 1.0))
                        dsum[pl.ds(j * 16, 16)] = inv
                    pltpu.sync_copy(dsum, invdeg_sp.at[pl.ds(g0, gch)])
                    @pl.when(c == 0)
                    def _():
                        pltpu.sync_copy(dsum, invdeg_out.at[pl.ds(g0, gch)])
                return carry
            lax.fori_loop(0, ngi, comb, 0)

        ebase = wid * ew

        def chunk(i, carry):
            b0 = ebase + i * k
            pltpu.sync_copy(src_hbm.at[pl.ds(b0, k)], src_v)
            pltpu.sync_copy(dst_hbm.at[pl.ds(b0, k)], dst_v)
            pltpu.async_copy(table_hbm.at[src_v], rows_v, sem).wait()
            pltpu.sync_copy(rows_v, acc.at[dst_v], add=True)
            return carry
        lax.fori_loop(0, nch, chunk, 0)
        plsc.subcore_barrier()

        def wback(t, carry):
            idx = s + t * _NS
            @pl.when(idx < nzc)
            def _():
                a0 = idx * zr
                pltpu.sync_copy(acc.at[pl.ds(a0, zr)], zrows)
                if layer1:
                    pltpu.sync_copy(invdeg_sp.at[pl.ds(a0, zr)], dchunk)
                else:
                    pltpu.sync_copy(invdeg_hbm.at[pl.ds(a0, zr)], dchunk)
                def srow(r, carry2):
                    rv = jnp.full((16,), r, jnp.int32)
                    inv = plsc.load_gather(dchunk, [rv])
                    for jj in range(d // 16):
                        zrows[r, pl.ds(jj * 16, 16)] = (
                            zrows[r, pl.ds(jj * 16, 16)] * inv)
                    return carry2
                lax.fori_loop(0, zr, srow, 0)
                pltpu.sync_copy(zrows, agg_out.at[pl.ds(c * n + a0, zr)])
            return carry
        lax.fori_loop(0, nzi, wback, 0)

    return pl.kernel(
        body, mesh=mesh, out_type=out_type, scratch_types=scratch,
        compiler_params=pltpu.CompilerParams(needs_layout_passes=False))


def _dense_body(relu, agg_a, agg_b, x_ref, wlt, blr, wrt, out_ref):
    mean = agg_a[...] + agg_b[...]
    y = (jnp.dot(mean, wlt[...], preferred_element_type=jnp.float32)
         + blr[...]
         + jnp.dot(x_ref[...], wrt[...], preferred_element_type=jnp.float32))
    out_ref[...] = jnp.maximum(y, 0.0) if relu else y


def _dense(agg2, x, wlt, blr, wrt, relu):
    n, d = x.shape
    r = 2000
    nb = n // r
    assert nb * r == n
    return pl.pallas_call(
        functools.partial(_dense_body, relu),
        grid=(nb,),
        in_specs=[
            pl.BlockSpec((r, d), lambda i: (i, 0)),
            pl.BlockSpec((r, d), lambda i, _nb=nb: (i + _nb, 0)),
            pl.BlockSpec((r, d), lambda i: (i, 0)),
            pl.BlockSpec((d, d), lambda i: (0, 0)),
            pl.BlockSpec((1, d), lambda i: (0, 0)),
            pl.BlockSpec((d, d), lambda i: (0, 0)),
        ],
        out_specs=pl.BlockSpec((r, d), lambda i: (i, 0)),
        out_shape=jax.ShapeDtypeStruct((n, d), jnp.float32),
    )(agg2, agg2, x, wlt, blr, wrt)


def kernel(x, edge_index, W1l, b1l, W1r, W2l, b2l, W2r):
    n, d = x.shape
    e = edge_index.shape[1]
    src = edge_index[0]
    dst = edge_index[1]

    seg1 = _make_segmean(n, d, e, layer1=True)
    seg2 = _make_segmean(n, d, e, layer1=False)

    mean1, invdeg = seg1(x, src, dst)
    h = _dense(mean1, x, W1l.T, b1l.reshape(1, -1), W1r.T, relu=True)
    (mean2,) = seg2(h, src, dst, invdeg)
    out = _dense(mean2, h, W2l.T, b2l.reshape(1, -1), W2r.T, relu=False)
    return out


# speedup vs baseline: 8.7037x; 1.6189x over previous
import functools

import jax
import jax.numpy as jnp
from jax import lax
from jax.experimental import pallas as pl
from jax.experimental.pallas import tpu as pltpu
from jax.experimental.pallas import tpu_sc as plsc

_NC = 2
_NS = 16


def _make_invdeg(n, e):
    ec = e // _NS
    ib = 2000
    ncc = ec // ib
    assert ncc * ib == ec
    gch = 80
    ngc = n // gch
    ngi = (ngc + _NS - 1) // _NS

    mesh = plsc.VectorSubcoreMesh(core_axis_name="c", subcore_axis_name="s")

    def body(dst_hbm, invdeg_out, cntbuf, degloc, sumbuf, dsum, degstage):
        c = lax.axis_index("c")
        s = lax.axis_index("s")

        def zdl(q, carry):
            degloc[pl.ds(q * 16, 16)] = jnp.zeros((16,), jnp.float32)
            return carry
        lax.fori_loop(0, n // 16, zdl, 0)
        ones16 = jnp.ones((16,), jnp.float32)
        cb = s * ec
        for cc in range(ncc):
            pltpu.sync_copy(dst_hbm.at[pl.ds(cb + cc * ib, ib)], cntbuf)
            def cnt(q, carry):
                iv = cntbuf[pl.ds(q * 16, 16)]
                plsc.addupdate_scatter(degloc, [iv], ones16)
                return carry
            lax.fori_loop(0, ib // 16, cnt, 0)
        pltpu.sync_copy(degloc, degstage.at[pl.ds(s * n, n)])
        plsc.subcore_barrier()

        def comb(t, carry):
            idx = s + t * _NS
            @pl.when((idx < ngc) & (c == 0))
            def _():
                g0 = idx * gch
                for a in range(_NS):
                    pltpu.sync_copy(degstage.at[pl.ds(a * n + g0, gch)],
                                    sumbuf.at[pl.ds(a * gch, gch)])
                for j in range(gch // 16):
                    tot = sumbuf[pl.ds(j * 16, 16)]
                    for a in range(1, _NS):
                        tot = tot + sumbuf[pl.ds(a * gch + j * 16, 16)]
                    inv = jnp.ones((16,), jnp.float32) / jnp.maximum(tot, 1.0)
                    dsum[pl.ds(j * 16, 16)] = inv
                pltpu.sync_copy(dsum, invdeg_out.at[pl.ds(g0, gch)])
            return carry
        lax.fori_loop(0, ngi, comb, 0)

    return pl.kernel(
        body, mesh=mesh,
        out_type=[jax.ShapeDtypeStruct((n,), jnp.float32)],
        scratch_types=[
            pltpu.VMEM((ib,), jnp.int32),
            pltpu.VMEM((n,), jnp.float32),
            pltpu.VMEM((_NS * gch,), jnp.float32),
            pltpu.VMEM((gch,), jnp.float32),
            pltpu.VMEM_SHARED((_NS * n,), jnp.float32),
        ],
        compiler_params=pltpu.CompilerParams(needs_layout_passes=False))


def _make_segmean(n, d, e):
    nw = _NC * _NS
    ew = e // nw
    k = 80
    ib = 2000
    ncb = ib // k
    nib = ew // ib
    assert ew * nw == e and ncb * k == ib and nib * ib == ew
    zr = 40
    nzc = n // zr
    assert zr * nzc == n
    nzi = (nzc + _NS - 1) // _NS

    mesh = plsc.VectorSubcoreMesh(core_axis_name="c", subcore_axis_name="s")

    def body(table_hbm, src_hbm, dst_hbm, invdeg_hbm, agg_out,
             srcall, dstall, src0, src1, dst0, dst1, rows0, rows1,
             zrows, dchunk, acc, gsem0, gsem1, ssem0, ssem1):
        src_v = (src0, src1)
        dst_v = (dst0, dst1)
        rows_v = (rows0, rows1)
        gsem = (gsem0, gsem1)
        ssem = (ssem0, ssem1)
        c = lax.axis_index("c")
        s = lax.axis_index("s")
        wid = c * _NS + s

        def zfill(r, carry):
            for jj in range(d // 16):
                zrows[r, pl.ds(jj * 16, 16)] = jnp.zeros((16,), jnp.float32)
            return carry
        lax.fori_loop(0, zr, zfill, 0)

        def zinit(t, carry):
            idx = s + t * _NS
            @pl.when(idx < nzc)
            def _():
                pltpu.sync_copy(zrows, acc.at[pl.ds(idx * zr, zr)])
            return carry
        lax.fori_loop(0, nzi, zinit, 0)
        plsc.subcore_barrier()

        ebase = wid * ew

        def fill_idx(b, i_chunk):
            off = i_chunk * k
            for j in range(k // 16):
                src_v[b][pl.ds(j * 16, 16)] = srcall[pl.ds(off + j * 16, 16)]
                dst_v[b][pl.ds(j * 16, 16)] = dstall[pl.ds(off + j * 16, 16)]

        def gather_start(b):
            pltpu.async_copy(table_hbm.at[src_v[b]], rows_v[b], gsem[b])

        def gather_wait(b):
            pltpu.make_async_copy(table_hbm.at[src_v[b]], rows_v[b],
                                  gsem[b]).wait()

        def scatter_start(b):
            pltpu.async_copy(rows_v[b], acc.at[dst_v[b]], ssem[b], add=True)

        def scatter_wait(b):
            pltpu.make_async_copy(rows_v[b], acc.at[dst_v[b]], ssem[b]).wait()

        def block(u, carry):
            bb = ebase + u * ib
            pltpu.sync_copy(src_hbm.at[pl.ds(bb, ib)], srcall)
            pltpu.sync_copy(dst_hbm.at[pl.ds(bb, ib)], dstall)
            fill_idx(0, 0)
            gather_start(0)

            def pair(p, carry2):
                for b in range(2):
                    i_ = 2 * p + b
                    @pl.when(i_ < ncb)
                    def _():
                        gather_wait(b)
                        nxt = i_ + 1
                        @pl.when(nxt < ncb)
                        def _():
                            @pl.when(nxt >= 2)
                            def _():
                                scatter_wait(1 - b)
                            fill_idx(1 - b, nxt)
                            gather_start(1 - b)
                        scatter_start(b)
                return carry2
            lax.fori_loop(0, (ncb + 1) // 2, pair, 0)
            scatter_wait((ncb - 2) % 2)
            scatter_wait((ncb - 1) % 2)
            return carry
        lax.fori_loop(0, nib, block, 0)
        plsc.subcore_barrier()

        def wback(t, carry):
            idx = s + t * _NS
            @pl.when(idx < nzc)
            def _():
                a0 = idx * zr
                pltpu.sync_copy(acc.at[pl.ds(a0, zr)], zrows)
                pltpu.sync_copy(invdeg_hbm.at[pl.ds(a0, zr)], dchunk)
                def srow(r, carry2):
                    rv = jnp.full((16,), r, jnp.int32)
                    inv = plsc.load_gather(dchunk, [rv])
                    for jj in range(d // 16):
                        zrows[r, pl.ds(jj * 16, 16)] = (
                            zrows[r, pl.ds(jj * 16, 16)] * inv)
                    return carry2
                lax.fori_loop(0, zr, srow, 0)
                pltpu.sync_copy(zrows, agg_out.at[pl.ds(c * n + a0, zr)])
            return carry
        lax.fori_loop(0, nzi, wback, 0)

    return pl.kernel(
        body, mesh=mesh,
        out_type=[jax.ShapeDtypeStruct((_NC * n, d), jnp.float32)],
        scratch_types=[
            pltpu.VMEM((2000,), jnp.int32),
            pltpu.VMEM((2000,), jnp.int32),
            pltpu.VMEM((80,), jnp.int32),
            pltpu.VMEM((80,), jnp.int32),
            pltpu.VMEM((80,), jnp.int32),
            pltpu.VMEM((80,), jnp.int32),
            pltpu.VMEM((80, d), jnp.float32),
            pltpu.VMEM((80, d), jnp.float32),
            pltpu.VMEM((40, d), jnp.float32),
            pltpu.VMEM((40,), jnp.float32),
            pltpu.VMEM_SHARED((n, d), jnp.float32),
            pltpu.SemaphoreType.DMA,
            pltpu.SemaphoreType.DMA,
            pltpu.SemaphoreType.DMA,
            pltpu.SemaphoreType.DMA,
        ],
        compiler_params=pltpu.CompilerParams(needs_layout_passes=False))


def _dense_body(relu, agg_a, agg_b, x_ref, wlt, blr, wrt, out_ref):
    mean = agg_a[...] + agg_b[...]
    y = (jnp.dot(mean, wlt[...], preferred_element_type=jnp.float32)
         + blr[...]
         + jnp.dot(x_ref[...], wrt[...], preferred_element_type=jnp.float32))
    out_ref[...] = jnp.maximum(y, 0.0) if relu else y


def _dense(agg2, x, wlt, blr, wrt, relu):
    n, d = x.shape
    r = 2000
    nb = n // r
    assert nb * r == n
    return pl.pallas_call(
        functools.partial(_dense_body, relu),
        grid=(nb,),
        in_specs=[
            pl.BlockSpec((r, d), lambda i: (i, 0)),
            pl.BlockSpec((r, d), lambda i, _nb=nb: (i + _nb, 0)),
            pl.BlockSpec((r, d), lambda i: (i, 0)),
            pl.BlockSpec((d, d), lambda i: (0, 0)),
            pl.BlockSpec((1, d), lambda i: (0, 0)),
            pl.BlockSpec((d, d), lambda i: (0, 0)),
        ],
        out_specs=pl.BlockSpec((r, d), lambda i: (i, 0)),
        out_shape=jax.ShapeDtypeStruct((n, d), jnp.float32),
    )(agg2, agg2, x, wlt, blr, wrt)


def kernel(x, edge_index, W1l, b1l, W1r, W2l, b2l, W2r):
    n, d = x.shape
    e = edge_index.shape[1]
    src = edge_index[0]
    dst = edge_index[1]

    invdeg_k = _make_invdeg(n, e)
    segmean = _make_segmean(n, d, e)

    (invdeg,) = invdeg_k(dst)
    (mean1,) = segmean(x, src, dst, invdeg)
    h = _dense(mean1, x, W1l.T, b1l.reshape(1, -1), W1r.T, relu=True)
    (mean2,) = segmean(h, src, dst, invdeg)
    out = _dense(mean2, h, W2l.T, b2l.reshape(1, -1), W2r.T, relu=False)
    return out


# speedup vs baseline: 11.7127x; 1.3457x over previous
import functools

import jax
import jax.numpy as jnp
from jax import lax
from jax.experimental import pallas as pl
from jax.experimental.pallas import tpu as pltpu
from jax.experimental.pallas import tpu_sc as plsc

_NC = 2
_NS = 16


def _make_invdeg(n, e):
    ec = e // _NS
    ib = 2000
    ncc = ec // ib
    assert ncc * ib == ec
    gch = 80
    ngc = n // gch
    ngi = (ngc + _NS - 1) // _NS

    mesh = plsc.VectorSubcoreMesh(core_axis_name="c", subcore_axis_name="s")

    def body(dst_hbm, invdeg_out, cntbuf, degloc, sumbuf, dsum, degstage):
        c = lax.axis_index("c")
        s = lax.axis_index("s")

        def zdl(q, carry):
            degloc[pl.ds(q * 16, 16)] = jnp.zeros((16,), jnp.float32)
            return carry
        lax.fori_loop(0, n // 16, zdl, 0)
        ones16 = jnp.ones((16,), jnp.float32)
        cb = s * ec
        for cc in range(ncc):
            pltpu.sync_copy(dst_hbm.at[pl.ds(cb + cc * ib, ib)], cntbuf)
            def cnt(q, carry):
                iv = cntbuf[pl.ds(q * 16, 16)]
                plsc.addupdate_scatter(degloc, [iv], ones16)
                return carry
            lax.fori_loop(0, ib // 16, cnt, 0)
        pltpu.sync_copy(degloc, degstage.at[pl.ds(s * n, n)])
        plsc.subcore_barrier()

        def comb(t, carry):
            idx = s + t * _NS
            @pl.when((idx < ngc) & (c == 0))
            def _():
                g0 = idx * gch
                for a in range(_NS):
                    pltpu.sync_copy(degstage.at[pl.ds(a * n + g0, gch)],
                                    sumbuf.at[pl.ds(a * gch, gch)])
                for j in range(gch // 16):
                    tot = sumbuf[pl.ds(j * 16, 16)]
                    for a in range(1, _NS):
                        tot = tot + sumbuf[pl.ds(a * gch + j * 16, 16)]
                    inv = jnp.ones((16,), jnp.float32) / jnp.maximum(tot, 1.0)
                    dsum[pl.ds(j * 16, 16)] = inv
                pltpu.sync_copy(dsum, invdeg_out.at[pl.ds(g0, gch)])
            return carry
        lax.fori_loop(0, ngi, comb, 0)

    return pl.kernel(
        body, mesh=mesh,
        out_type=[jax.ShapeDtypeStruct((n,), jnp.float32)],
        scratch_types=[
            pltpu.VMEM((ib,), jnp.int32),
            pltpu.VMEM((n,), jnp.float32),
            pltpu.VMEM((_NS * gch,), jnp.float32),
            pltpu.VMEM((gch,), jnp.float32),
            pltpu.VMEM_SHARED((_NS * n,), jnp.float32),
        ],
        compiler_params=pltpu.CompilerParams(needs_layout_passes=False))


def _make_segmean(n, d, e):
    nw = _NC * _NS
    ew = e // nw
    k = 80
    ib = 2000
    ncb = ib // k
    nib = ew // ib
    assert ew * nw == e and ncb * k == ib and nib * ib == ew
    zr = 40
    nzc = n // zr
    assert zr * nzc == n
    nzi = (nzc + _NS - 1) // _NS

    mesh = plsc.VectorSubcoreMesh(core_axis_name="c", subcore_axis_name="s")

    def body(table_hbm, src_hbm, dst_hbm, invdeg_hbm, agg_out,
             srcall, dstall, src0, src1, src2, dst0, dst1, dst2,
             rows0, rows1, rows2, zrows, dchunk, acc,
             gsem0, gsem1, gsem2, ssem0, ssem1, ssem2):
        src_v = (src0, src1, src2)
        dst_v = (dst0, dst1, dst2)
        rows_v = (rows0, rows1, rows2)
        gsem = (gsem0, gsem1, gsem2)
        ssem = (ssem0, ssem1, ssem2)
        c = lax.axis_index("c")
        s = lax.axis_index("s")
        wid = c * _NS + s

        def zfill(r, carry):
            for jj in range(d // 16):
                zrows[r, pl.ds(jj * 16, 16)] = jnp.zeros((16,), jnp.float32)
            return carry
        lax.fori_loop(0, zr, zfill, 0)

        def zinit(t, carry):
            idx = s + t * _NS
            @pl.when(idx < nzc)
            def _():
                pltpu.sync_copy(zrows, acc.at[pl.ds(idx * zr, zr)])
            return carry
        lax.fori_loop(0, nzi, zinit, 0)
        plsc.subcore_barrier()

        ebase = wid * ew

        def fill_idx(b, i_chunk):
            off = i_chunk * k
            for j in range(k // 16):
                src_v[b][pl.ds(j * 16, 16)] = srcall[pl.ds(off + j * 16, 16)]
                dst_v[b][pl.ds(j * 16, 16)] = dstall[pl.ds(off + j * 16, 16)]

        def gather_start(b):
            pltpu.async_copy(table_hbm.at[src_v[b]], rows_v[b], gsem[b])

        def gather_wait(b):
            pltpu.make_async_copy(table_hbm.at[src_v[b]], rows_v[b],
                                  gsem[b]).wait()

        def scatter_start(b):
            pltpu.async_copy(rows_v[b], acc.at[dst_v[b]], ssem[b], add=True)

        def scatter_wait(b):
            pltpu.make_async_copy(rows_v[b], acc.at[dst_v[b]], ssem[b]).wait()

        def block(u, carry):
            bb = ebase + u * ib
            pltpu.sync_copy(src_hbm.at[pl.ds(bb, ib)], srcall)
            pltpu.sync_copy(dst_hbm.at[pl.ds(bb, ib)], dstall)
            fill_idx(0, 0)
            gather_start(0)
            fill_idx(1, 1)
            gather_start(1)

            def tri(p, carry2):
                for b in range(3):
                    i_ = 3 * p + b
                    @pl.when(i_ < ncb)
                    def _():
                        gather_wait(b)
                        nxt = i_ + 2
                        @pl.when(nxt < ncb)
                        def _():
                            nb2 = (b + 2) % 3
                            @pl.when(i_ >= 1)
                            def _():
                                scatter_wait(nb2)
                            fill_idx(nb2, nxt)
                            gather_start(nb2)
                        scatter_start(b)
                return carry2
            lax.fori_loop(0, (ncb + 2) // 3, tri, 0)
            scatter_wait((ncb - 3) % 3)
            scatter_wait((ncb - 2) % 3)
            scatter_wait((ncb - 1) % 3)
            return carry
        lax.fori_loop(0, nib, block, 0)
        plsc.subcore_barrier()

        def wback(t, carry):
            idx = s + t * _NS
            @pl.when(idx < nzc)
            def _():
                a0 = idx * zr
                pltpu.sync_copy(acc.at[pl.ds(a0, zr)], zrows)
                pltpu.sync_copy(invdeg_hbm.at[pl.ds(a0, zr)], dchunk)
                def srow(r, carry2):
                    rv = jnp.full((16,), r, jnp.int32)
                    inv = plsc.load_gather(dchunk, [rv])
                    for jj in range(d // 16):
                        zrows[r, pl.ds(jj * 16, 16)] = (
                            zrows[r, pl.ds(jj * 16, 16)] * inv)
                    return carry2
                lax.fori_loop(0, zr, srow, 0)
                pltpu.sync_copy(zrows, agg_out.at[pl.ds(c * n + a0, zr)])
            return carry
        lax.fori_loop(0, nzi, wback, 0)

    return pl.kernel(
        body, mesh=mesh,
        out_type=[jax.ShapeDtypeStruct((_NC * n, d), jnp.float32)],
        scratch_types=[
            pltpu.VMEM((2000,), jnp.int32),
            pltpu.VMEM((2000,), jnp.int32),
            pltpu.VMEM((80,), jnp.int32),
            pltpu.VMEM((80,), jnp.int32),
            pltpu.VMEM((80,), jnp.int32),
            pltpu.VMEM((80,), jnp.int32),
            pltpu.VMEM((80,), jnp.int32),
            pltpu.VMEM((80,), jnp.int32),
            pltpu.VMEM((80, d), jnp.float32),
            pltpu.VMEM((80, d), jnp.float32),
            pltpu.VMEM((80, d), jnp.float32),
            pltpu.VMEM((40, d), jnp.float32),
            pltpu.VMEM((40,), jnp.float32),
            pltpu.VMEM_SHARED((n, d), jnp.float32),
            pltpu.SemaphoreType.DMA,
            pltpu.SemaphoreType.DMA,
            pltpu.SemaphoreType.DMA,
            pltpu.SemaphoreType.DMA,
            pltpu.SemaphoreType.DMA,
            pltpu.SemaphoreType.DMA,
        ],
        compiler_params=pltpu.CompilerParams(needs_layout_passes=False))


def _dense_body(relu, agg_a, agg_b, x_ref, wlt, blr, wrt, out_ref):
    mean = agg_a[...] + agg_b[...]
    y = (jnp.dot(mean, wlt[...], preferred_element_type=jnp.float32)
         + blr[...]
         + jnp.dot(x_ref[...], wrt[...], preferred_element_type=jnp.float32))
    out_ref[...] = jnp.maximum(y, 0.0) if relu else y


def _dense(agg2, x, wlt, blr, wrt, relu):
    n, d = x.shape
    r = 2000
    nb = n // r
    assert nb * r == n
    return pl.pallas_call(
        functools.partial(_dense_body, relu),
        grid=(nb,),
        in_specs=[
            pl.BlockSpec((r, d), lambda i: (i, 0)),
            pl.BlockSpec((r, d), lambda i, _nb=nb: (i + _nb, 0)),
            pl.BlockSpec((r, d), lambda i: (i, 0)),
            pl.BlockSpec((d, d), lambda i: (0, 0)),
            pl.BlockSpec((1, d), lambda i: (0, 0)),
            pl.BlockSpec((d, d), lambda i: (0, 0)),
        ],
        out_specs=pl.BlockSpec((r, d), lambda i: (i, 0)),
        out_shape=jax.ShapeDtypeStruct((n, d), jnp.float32),
    )(agg2, agg2, x, wlt, blr, wrt)


def kernel(x, edge_index, W1l, b1l, W1r, W2l, b2l, W2r):
    n, d = x.shape
    e = edge_index.shape[1]
    src = edge_index[0]
    dst = edge_index[1]

    invdeg_k = _make_invdeg(n, e)
    segmean = _make_segmean(n, d, e)

    (invdeg,) = invdeg_k(dst)
    (mean1,) = segmean(x, src, dst, invdeg)
    h = _dense(mean1, x, W1l.T, b1l.reshape(1, -1), W1r.T, relu=True)
    (mean2,) = segmean(h, src, dst, invdeg)
    out = _dense(mean2, h, W2l.T, b2l.reshape(1, -1), W2r.T, relu=False)
    return out


# speedup vs baseline: 12.1601x; 1.0382x over previous
import functools

import jax
import jax.numpy as jnp
from jax import lax
from jax.experimental import pallas as pl
from jax.experimental.pallas import tpu as pltpu
from jax.experimental.pallas import tpu_sc as plsc

_NC = 2
_NS = 16


def _make_invdeg(n, e):
    ec = e // _NS
    ib = 2000
    ncc = ec // ib
    assert ncc * ib == ec
    gch = 80
    ngc = n // gch
    ngi = (ngc + _NS - 1) // _NS

    mesh = plsc.VectorSubcoreMesh(core_axis_name="c", subcore_axis_name="s")

    def body(dst_hbm, invdeg_out, cntbuf, degloc, sumbuf, dsum, degstage,
             csem):
        c = lax.axis_index("c")
        s = lax.axis_index("s")

        def zdl(q, carry):
            degloc[pl.ds(q * 16, 16)] = jnp.zeros((16,), jnp.float32)
            return carry
        lax.fori_loop(0, n // 16, zdl, 0)
        ones16 = jnp.ones((16,), jnp.float32)
        cb = s * ec
        for cc in range(ncc):
            pltpu.sync_copy(dst_hbm.at[pl.ds(cb + cc * ib, ib)], cntbuf)
            def cnt(q, carry):
                iv = cntbuf[pl.ds(q * 16, 16)]
                plsc.addupdate_scatter(degloc, [iv], ones16)
                return carry
            lax.fori_loop(0, ib // 16, cnt, 0)
        pltpu.sync_copy(degloc, degstage.at[pl.ds(s * n, n)])
        plsc.subcore_barrier()

        def comb(t, carry):
            idx = s + t * _NS
            @pl.when((idx < ngc) & (c == 0))
            def _():
                g0 = idx * gch
                for a in range(_NS):
                    pltpu.async_copy(degstage.at[pl.ds(a * n + g0, gch)],
                                     sumbuf.at[pl.ds(a * gch, gch)], csem)
                for a in range(_NS):
                    pltpu.make_async_copy(
                        degstage.at[pl.ds(a * n + g0, gch)],
                        sumbuf.at[pl.ds(a * gch, gch)], csem).wait()
                for j in range(gch // 16):
                    tot = sumbuf[pl.ds(j * 16, 16)]
                    for a in range(1, _NS):
                        tot = tot + sumbuf[pl.ds(a * gch + j * 16, 16)]
                    inv = jnp.ones((16,), jnp.float32) / jnp.maximum(tot, 1.0)
                    dsum[pl.ds(j * 16, 16)] = inv
                pltpu.sync_copy(dsum, invdeg_out.at[pl.ds(g0, gch)])
            return carry
        lax.fori_loop(0, ngi, comb, 0)

    return pl.kernel(
        body, mesh=mesh,
        out_type=[jax.ShapeDtypeStruct((n,), jnp.float32)],
        scratch_types=[
            pltpu.VMEM((ib,), jnp.int32),
            pltpu.VMEM((n,), jnp.float32),
            pltpu.VMEM((_NS * gch,), jnp.float32),
            pltpu.VMEM((gch,), jnp.float32),
            pltpu.VMEM_SHARED((_NS * n,), jnp.float32),
            pltpu.SemaphoreType.DMA,
        ],
        compiler_params=pltpu.CompilerParams(needs_layout_passes=False))


def _make_segmean(n, d, e):
    nw = _NC * _NS
    ew = e // nw
    k = 80
    ib = 2000
    ncb = ib // k
    nib = ew // ib
    assert ew * nw == e and ncb * k == ib and nib * ib == ew
    zr = 40
    nzc = n // zr
    assert zr * nzc == n
    nzi = (nzc + _NS - 1) // _NS

    mesh = plsc.VectorSubcoreMesh(core_axis_name="c", subcore_axis_name="s")

    def body(table_hbm, src_hbm, dst_hbm, invdeg_hbm, agg_out,
             srcall, dstall, src0, src1, src2, dst0, dst1, dst2,
             rows0, rows1, rows2, zrows, dchunk, acc,
             gsem0, gsem1, gsem2, ssem0, ssem1, ssem2):
        src_v = (src0, src1, src2)
        dst_v = (dst0, dst1, dst2)
        rows_v = (rows0, rows1, rows2)
        gsem = (gsem0, gsem1, gsem2)
        ssem = (ssem0, ssem1, ssem2)
        c = lax.axis_index("c")
        s = lax.axis_index("s")
        wid = c * _NS + s

        def zfill(r, carry):
            for jj in range(d // 16):
                rows_v[0][r, pl.ds(jj * 16, 16)] = jnp.zeros((16,),
                                                             jnp.float32)
            return carry
        lax.fori_loop(0, k, zfill, 0)
        nz2 = n // k

        def zinit(t, carry):
            idx = s + t * _NS
            @pl.when(idx < nz2)
            def _():
                pltpu.sync_copy(rows_v[0], acc.at[pl.ds(idx * k, k)])
            return carry
        lax.fori_loop(0, (nz2 + _NS - 1) // _NS, zinit, 0)
        plsc.subcore_barrier()

        ebase = wid * ew

        def fill_idx(b, i_chunk):
            off = i_chunk * k
            for j in range(k // 16):
                src_v[b][pl.ds(j * 16, 16)] = srcall[pl.ds(off + j * 16, 16)]
                dst_v[b][pl.ds(j * 16, 16)] = dstall[pl.ds(off + j * 16, 16)]

        def gather_start(b):
            pltpu.async_copy(table_hbm.at[src_v[b]], rows_v[b], gsem[b])

        def gather_wait(b):
            pltpu.make_async_copy(table_hbm.at[src_v[b]], rows_v[b],
                                  gsem[b]).wait()

        def scatter_start(b):
            pltpu.async_copy(rows_v[b], acc.at[dst_v[b]], ssem[b], add=True)

        def scatter_wait(b):
            pltpu.make_async_copy(rows_v[b], acc.at[dst_v[b]], ssem[b]).wait()

        def block(u, carry):
            bb = ebase + u * ib
            pltpu.sync_copy(src_hbm.at[pl.ds(bb, ib)], srcall)
            pltpu.sync_copy(dst_hbm.at[pl.ds(bb, ib)], dstall)
            fill_idx(0, 0)
            gather_start(0)
            fill_idx(1, 1)
            gather_start(1)

            def tri(p, carry2):
                for b in range(3):
                    i_ = 3 * p + b
                    @pl.when(i_ < ncb)
                    def _():
                        gather_wait(b)
                        nxt = i_ + 2
                        @pl.when(nxt < ncb)
                        def _():
                            nb2 = (b + 2) % 3
                            @pl.when(i_ >= 1)
                            def _():
                                scatter_wait(nb2)
                            fill_idx(nb2, nxt)
                            gather_start(nb2)
                        scatter_start(b)
                return carry2
            lax.fori_loop(0, (ncb + 2) // 3, tri, 0)
            scatter_wait((ncb - 3) % 3)
            scatter_wait((ncb - 2) % 3)
            scatter_wait((ncb - 1) % 3)
            return carry
        lax.fori_loop(0, nib, block, 0)
        plsc.subcore_barrier()

        def wback(t, carry):
            idx = s + t * _NS
            @pl.when(idx < nzc)
            def _():
                a0 = idx * zr
                @pl.when(t >= 1)
                def _():
                    pltpu.make_async_copy(
                        zrows, agg_out.at[pl.ds(c * n, zr)], ssem[0]).wait()
                pltpu.sync_copy(acc.at[pl.ds(a0, zr)], zrows)
                pltpu.sync_copy(invdeg_hbm.at[pl.ds(a0, zr)], dchunk)
                def srow(r, carry2):
                    rv = jnp.full((16,), r, jnp.int32)
                    inv = plsc.load_gather(dchunk, [rv])
                    for jj in range(d // 16):
                        zrows[r, pl.ds(jj * 16, 16)] = (
                            zrows[r, pl.ds(jj * 16, 16)] * inv)
                    return carry2
                lax.fori_loop(0, zr, srow, 0)
                pltpu.async_copy(zrows, agg_out.at[pl.ds(c * n + a0, zr)],
                                 ssem[0])
            return carry
        lax.fori_loop(0, nzi, wback, 0)
        pltpu.make_async_copy(zrows, agg_out.at[pl.ds(c * n, zr)],
                              ssem[0]).wait()

    return pl.kernel(
        body, mesh=mesh,
        out_type=[jax.ShapeDtypeStruct((_NC * n, d), jnp.float32)],
        scratch_types=[
            pltpu.VMEM((2000,), jnp.int32),
            pltpu.VMEM((2000,), jnp.int32),
            pltpu.VMEM((80,), jnp.int32),
            pltpu.VMEM((80,), jnp.int32),
            pltpu.VMEM((80,), jnp.int32),
            pltpu.VMEM((80,), jnp.int32),
            pltpu.VMEM((80,), jnp.int32),
            pltpu.VMEM((80,), jnp.int32),
            pltpu.VMEM((80, d), jnp.float32),
            pltpu.VMEM((80, d), jnp.float32),
            pltpu.VMEM((80, d), jnp.float32),
            pltpu.VMEM((40, d), jnp.float32),
            pltpu.VMEM((40,), jnp.float32),
            pltpu.VMEM_SHARED((n, d), jnp.float32),
            pltpu.SemaphoreType.DMA,
            pltpu.SemaphoreType.DMA,
            pltpu.SemaphoreType.DMA,
            pltpu.SemaphoreType.DMA,
            pltpu.SemaphoreType.DMA,
            pltpu.SemaphoreType.DMA,
        ],
        compiler_params=pltpu.CompilerParams(needs_layout_passes=False))


def _dense_body(relu, agg_a, agg_b, x_ref, wlt, blr, wrt, out_ref):
    mean = agg_a[...] + agg_b[...]
    y = (jnp.dot(mean, wlt[...], preferred_element_type=jnp.float32)
         + blr[...]
         + jnp.dot(x_ref[...], wrt[...], preferred_element_type=jnp.float32))
    out_ref[...] = jnp.maximum(y, 0.0) if relu else y


def _dense(agg2, x, wlt, blr, wrt, relu):
    n, d = x.shape
    r = 2000
    nb = n // r
    assert nb * r == n
    return pl.pallas_call(
        functools.partial(_dense_body, relu),
        grid=(nb,),
        in_specs=[
            pl.BlockSpec((r, d), lambda i: (i, 0)),
            pl.BlockSpec((r, d), lambda i, _nb=nb: (i + _nb, 0)),
            pl.BlockSpec((r, d), lambda i: (i, 0)),
            pl.BlockSpec((d, d), lambda i: (0, 0)),
            pl.BlockSpec((1, d), lambda i: (0, 0)),
            pl.BlockSpec((d, d), lambda i: (0, 0)),
        ],
        out_specs=pl.BlockSpec((r, d), lambda i: (i, 0)),
        out_shape=jax.ShapeDtypeStruct((n, d), jnp.float32),
    )(agg2, agg2, x, wlt, blr, wrt)


def kernel(x, edge_index, W1l, b1l, W1r, W2l, b2l, W2r):
    n, d = x.shape
    e = edge_index.shape[1]
    src = edge_index[0]
    dst = edge_index[1]

    invdeg_k = _make_invdeg(n, e)
    segmean = _make_segmean(n, d, e)

    (invdeg,) = invdeg_k(dst)
    (mean1,) = segmean(x, src, dst, invdeg)
    h = _dense(mean1, x, W1l.T, b1l.reshape(1, -1), W1r.T, relu=True)
    (mean2,) = segmean(h, src, dst, invdeg)
    out = _dense(mean2, h, W2l.T, b2l.reshape(1, -1), W2r.T, relu=False)
    return out
